# Initial kernel scaffold; baseline (speedup 1.0000x reference)
import jax
import jax.numpy as jnp
from jax.experimental import pallas as pl


def kernel(inputs, table, bias, gamma, beta):
    raise NotImplementedError("write your pallas kernel here")


# speedup vs baseline: 7.5355x; 7.5355x over previous
import functools

import jax
import jax.numpy as jnp
from jax import lax
from jax.experimental import pallas as pl
from jax.experimental.pallas import tpu as pltpu
from jax.experimental.pallas import tpu_sc as plsc

VOCAB = 100000
HIDDEN = 128
BATCH = 4096
SEQ = 50
EPS = 1e-3

NC = 2
NS = 16
NW = NC * NS
RPW = BATCH // NW
CHUNK = 8
NCHUNK = RPW // CHUNK
IDX_PER_CHUNK = CHUNK * SEQ
GATHER_GROUPS = 4
IDX_PER_GATHER = IDX_PER_CHUNK // GATHER_GROUPS
LANES = 16
HCHUNKS = HIDDEN // LANES


def _make_pool_kernel():
    mesh = plsc.VectorSubcoreMesh(core_axis_name="c", subcore_axis_name="s")

    @functools.partial(
        pl.kernel,
        mesh=mesh,
        out_type=jax.ShapeDtypeStruct((BATCH, HIDDEN), jnp.float32),
        scratch_types=[
            pltpu.VMEM((GATHER_GROUPS, IDX_PER_GATHER), jnp.int32),
            pltpu.VMEM((IDX_PER_CHUNK, HIDDEN), jnp.float32),
            pltpu.VMEM((CHUNK, HIDDEN), jnp.float32),
            pltpu.SemaphoreType.DMA,
        ],
    )
    def pool(idx_hbm, table_hbm, out_hbm, idx_v, rows_v, pooled_v, sem):
        wid = lax.axis_index("s") * NC + lax.axis_index("c")

        def chunk_body(c, carry):
            idx_row = wid * (NCHUNK * GATHER_GROUPS) + c * GATHER_GROUPS
            pltpu.sync_copy(idx_hbm.at[pl.ds(idx_row, GATHER_GROUPS)], idx_v)
            cps = [
                pltpu.async_copy(
                    table_hbm.at[idx_v.at[i]],
                    rows_v.at[pl.ds(i * IDX_PER_GATHER, IDX_PER_GATHER)],
                    sem,
                )
                for i in range(GATHER_GROUPS)
            ]
            for cp in cps:
                cp.wait()

            def row_body(r, carry2):
                base = r * SEQ
                for hg in range(0, HCHUNKS, 4):
                    accs = [rows_v[base, pl.ds((hg + k) * LANES, LANES)] for k in range(4)]
                    for j in range(1, SEQ):
                        for k in range(4):
                            accs[k] = accs[k] + rows_v[base + j, pl.ds((hg + k) * LANES, LANES)]
                    for k in range(4):
                        pooled_v[r, pl.ds((hg + k) * LANES, LANES)] = accs[k] * (1.0 / SEQ)
                return carry2

            lax.fori_loop(0, CHUNK, row_body, 0)
            pltpu.sync_copy(pooled_v, out_hbm.at[pl.ds(wid * RPW + c * CHUNK, CHUNK)])
            return carry

        lax.fori_loop(0, NCHUNK, chunk_body, 0)

    return pool


_pool = _make_pool_kernel()


def _ln_body(x_ref, b_ref, g_ref, bt_ref, o_ref):
    x = x_ref[...] + b_ref[...]
    mu = jnp.mean(x, axis=0, keepdims=True)
    xc = x - mu
    var = jnp.mean(xc * xc, axis=0, keepdims=True)
    o_ref[...] = xc * lax.rsqrt(var + EPS) * g_ref[...] + bt_ref[...]


def kernel(inputs, table, bias, gamma, beta):
    idx2d = inputs.reshape(BATCH * SEQ // IDX_PER_GATHER, IDX_PER_GATHER)
    pooled = _pool(idx2d, table)
    out = pl.pallas_call(
        _ln_body,
        out_shape=jax.ShapeDtypeStruct((BATCH, HIDDEN), jnp.float32),
    )(pooled, bias.reshape(1, HIDDEN), gamma.reshape(BATCH, 1), beta.reshape(BATCH, 1))
    return out


# speedup vs baseline: 10.8219x; 1.4361x over previous
import functools

import jax
import jax.numpy as jnp
from jax import lax
from jax.experimental import pallas as pl
from jax.experimental.pallas import tpu as pltpu
from jax.experimental.pallas import tpu_sc as plsc

VOCAB = 100000
HIDDEN = 128
BATCH = 4096
SEQ = 50
EPS = 1e-3

NC = 2
NS = 16
NW = NC * NS
RPW = BATCH // NW
CHUNK = 8
NCHUNK = RPW // CHUNK
IDX_PER_CHUNK = CHUNK * SEQ
GATHER_GROUPS = 4
IDX_PER_GATHER = IDX_PER_CHUNK // GATHER_GROUPS
LANES = 16
HCHUNKS = HIDDEN // LANES


def _make_pool_kernel():
    mesh = plsc.VectorSubcoreMesh(core_axis_name="c", subcore_axis_name="s")

    @functools.partial(
        pl.kernel,
        mesh=mesh,
        out_type=jax.ShapeDtypeStruct((BATCH, HIDDEN), jnp.float32),
        scratch_types=[
            pltpu.VMEM((2, GATHER_GROUPS, IDX_PER_GATHER), jnp.int32),
            pltpu.VMEM((2, IDX_PER_CHUNK, HIDDEN), jnp.float32),
            pltpu.VMEM((CHUNK, HIDDEN), jnp.float32),
            pltpu.SemaphoreType.DMA,
            pltpu.SemaphoreType.DMA,
        ],
    )
    def pool(idx_hbm, table_hbm, out_hbm, idx_v, rows_v, pooled_v, sem0, sem1):
        wid = lax.axis_index("s") * NC + lax.axis_index("c")
        sems = (sem0, sem1)

        def issue(p, c):
            idx_row = wid * (NCHUNK * GATHER_GROUPS) + c * GATHER_GROUPS
            pltpu.sync_copy(idx_hbm.at[pl.ds(idx_row, GATHER_GROUPS)], idx_v.at[p])
            for i in range(GATHER_GROUPS):
                pltpu.async_copy(
                    table_hbm.at[idx_v.at[p, i]],
                    rows_v.at[p, pl.ds(i * IDX_PER_GATHER, IDX_PER_GATHER)],
                    sems[p],
                )

        def wait_buf(p):
            for i in range(GATHER_GROUPS):
                pltpu.make_async_copy(
                    table_hbm.at[idx_v.at[p, i]],
                    rows_v.at[p, pl.ds(i * IDX_PER_GATHER, IDX_PER_GATHER)],
                    sems[p],
                ).wait()

        def compute(p, c):
            def row_body(r, carry2):
                base = r * SEQ
                for hg in range(0, HCHUNKS, 4):
                    accs = [rows_v[p, base, pl.ds((hg + k) * LANES, LANES)] for k in range(4)]
                    for j in range(1, SEQ):
                        for k in range(4):
                            accs[k] = accs[k] + rows_v[p, base + j, pl.ds((hg + k) * LANES, LANES)]
                    for k in range(4):
                        pooled_v[r, pl.ds((hg + k) * LANES, LANES)] = accs[k] * (1.0 / SEQ)
                return carry2

            lax.fori_loop(0, CHUNK, row_body, 0)
            pltpu.sync_copy(pooled_v, out_hbm.at[pl.ds(wid * RPW + c * CHUNK, CHUNK)])

        issue(0, 0)

        def g_body(g, carry):
            c0 = 2 * g
            issue(1, c0 + 1)
            wait_buf(0)
            compute(0, c0)

            @pl.when(g < NCHUNK // 2 - 1)
            def _():
                issue(0, c0 + 2)

            wait_buf(1)
            compute(1, c0 + 1)
            return carry

        lax.fori_loop(0, NCHUNK // 2, g_body, 0)

    return pool


_pool = _make_pool_kernel()


def _ln_body(x_ref, b_ref, g_ref, bt_ref, o_ref):
    x = x_ref[...] + b_ref[...]
    mu = jnp.mean(x, axis=0, keepdims=True)
    xc = x - mu
    var = jnp.mean(xc * xc, axis=0, keepdims=True)
    o_ref[...] = xc * lax.rsqrt(var + EPS) * g_ref[...] + bt_ref[...]


def kernel(inputs, table, bias, gamma, beta):
    idx2d = inputs.reshape(BATCH * SEQ // IDX_PER_GATHER, IDX_PER_GATHER)
    pooled = _pool(idx2d, table)
    out = pl.pallas_call(
        _ln_body,
        out_shape=jax.ShapeDtypeStruct((BATCH, HIDDEN), jnp.float32),
    )(pooled, bias.reshape(1, HIDDEN), gamma.reshape(BATCH, 1), beta.reshape(BATCH, 1))
    return out


# speedup vs baseline: 12.0686x; 1.1152x over previous
import functools

import jax
import jax.numpy as jnp
from jax import lax
from jax.experimental import pallas as pl
from jax.experimental.pallas import tpu as pltpu
from jax.experimental.pallas import tpu_sc as plsc

VOCAB = 100000
HIDDEN = 128
BATCH = 4096
SEQ = 50
EPS = 1e-3

NC = 2
NS = 16
NW = NC * NS
RPW = BATCH // NW
CHUNK = 8
NCHUNK = RPW // CHUNK
IDX_PER_CHUNK = CHUNK * SEQ
GATHER_GROUPS = 4
IDX_PER_GATHER = IDX_PER_CHUNK // GATHER_GROUPS
LANES = 16
HCHUNKS = HIDDEN // LANES


def _make_pool_kernel():
    mesh = plsc.VectorSubcoreMesh(core_axis_name="c", subcore_axis_name="s")

    @functools.partial(
        pl.kernel,
        mesh=mesh,
        out_type=jax.ShapeDtypeStruct((BATCH, HIDDEN), jnp.float32),
        scratch_types=[
            pltpu.VMEM((2, GATHER_GROUPS, IDX_PER_GATHER), jnp.int32),
            pltpu.VMEM((2, IDX_PER_CHUNK, HIDDEN), jnp.float32),
            pltpu.VMEM((CHUNK, HIDDEN), jnp.float32),
            pltpu.SemaphoreType.DMA,
            pltpu.SemaphoreType.DMA,
        ],
    )
    def pool(idx_hbm, table_hbm, out_hbm, idx_v, rows_v, pooled_v, sem0, sem1):
        wid = lax.axis_index("s") * NC + lax.axis_index("c")
        sems = (sem0, sem1)

        def issue(p, c):
            idx_row = wid * (NCHUNK * GATHER_GROUPS) + c * GATHER_GROUPS
            pltpu.sync_copy(idx_hbm.at[pl.ds(idx_row, GATHER_GROUPS)], idx_v.at[p])
            for i in range(GATHER_GROUPS):
                pltpu.async_copy(
                    table_hbm.at[idx_v.at[p, i]],
                    rows_v.at[p, pl.ds(i * IDX_PER_GATHER, IDX_PER_GATHER)],
                    sems[p],
                )

        def wait_buf(p):
            for i in range(GATHER_GROUPS):
                pltpu.make_async_copy(
                    table_hbm.at[idx_v.at[p, i]],
                    rows_v.at[p, pl.ds(i * IDX_PER_GATHER, IDX_PER_GATHER)],
                    sems[p],
                ).wait()

        def compute(p, c):
            def row_body(r, carry2):
                base = r * SEQ
                for hg, width in ((0, 3), (3, 3), (6, 2)):
                    accs = [rows_v[p, base, pl.ds((hg + k) * LANES, LANES)] for k in range(width)]
                    for j in range(1, SEQ):
                        for k in range(width):
                            accs[k] = accs[k] + rows_v[p, base + j, pl.ds((hg + k) * LANES, LANES)]
                    for k in range(width):
                        pooled_v[r, pl.ds((hg + k) * LANES, LANES)] = accs[k] * (1.0 / SEQ)
                return carry2

            lax.fori_loop(0, CHUNK, row_body, 0)
            pltpu.sync_copy(pooled_v, out_hbm.at[pl.ds(wid * RPW + c * CHUNK, CHUNK)])

        issue(0, 0)

        def g_body(g, carry):
            c0 = 2 * g
            issue(1, c0 + 1)
            wait_buf(0)
            compute(0, c0)

            @pl.when(g < NCHUNK // 2 - 1)
            def _():
                issue(0, c0 + 2)

            wait_buf(1)
            compute(1, c0 + 1)
            return carry

        lax.fori_loop(0, NCHUNK // 2, g_body, 0)

    return pool


_pool = _make_pool_kernel()


def _ln_body(x_ref, b_ref, g_ref, bt_ref, o_ref):
    x = x_ref[...] + b_ref[...]
    mu = jnp.mean(x, axis=0, keepdims=True)
    xc = x - mu
    var = jnp.mean(xc * xc, axis=0, keepdims=True)
    o_ref[...] = xc * lax.rsqrt(var + EPS) * g_ref[...] + bt_ref[...]


def kernel(inputs, table, bias, gamma, beta):
    idx2d = inputs.reshape(BATCH * SEQ // IDX_PER_GATHER, IDX_PER_GATHER)
    pooled = _pool(idx2d, table)
    out = pl.pallas_call(
        _ln_body,
        out_shape=jax.ShapeDtypeStruct((BATCH, HIDDEN), jnp.float32),
    )(pooled, bias.reshape(1, HIDDEN), gamma.reshape(BATCH, 1), beta.reshape(BATCH, 1))
    return out
